# Initial kernel scaffold; baseline (speedup 1.0000x reference)
import jax
import jax.numpy as jnp
from jax.experimental import pallas as pl


def kernel(input_batch, table):
    raise NotImplementedError("write your pallas kernel here")


# speedup vs baseline: 2.1812x; 2.1812x over previous
import functools

import jax
import jax.numpy as jnp
from jax import lax
from jax.experimental import pallas as pl
from jax.experimental.pallas import tpu as pltpu
from jax.experimental.pallas import tpu_sc as plsc

NC = 2
NS = 16
NW = NC * NS
D = 1024
C = 64


@functools.lru_cache(maxsize=None)
def _make(B):
    bpw = B // NW
    nchunks = bpw // C
    mesh = plsc.VectorSubcoreMesh(core_axis_name="c", subcore_axis_name="s")

    @functools.partial(
        pl.kernel,
        mesh=mesh,
        out_type=jax.ShapeDtypeStruct((B, D), jnp.float32),
        scratch_types=[
            pltpu.VMEM((bpw,), jnp.int32),
            pltpu.VMEM((C, D), jnp.float32),
            pltpu.SemaphoreType.DMA,
        ],
    )
    def gather_kernel(table_hbm, idx_hbm, out_hbm, idx_v, rows_v, sem):
        wid = lax.axis_index("s") * NC + lax.axis_index("c")
        base = wid * bpw
        pltpu.sync_copy(idx_hbm.at[pl.ds(base, bpw)], idx_v)

        def chunk(i, carry):
            off = i * C
            pltpu.async_copy(
                table_hbm.at[idx_v.at[pl.ds(off, C)]], rows_v, sem
            ).wait()
            pltpu.sync_copy(rows_v, out_hbm.at[pl.ds(base + off, C)])
            return carry

        lax.fori_loop(0, nchunks, chunk, 0, unroll=False)

    return gather_kernel


def kernel(input_batch, table):
    shape = input_batch.shape
    idx = input_batch.reshape(-1).astype(jnp.int32)
    out = _make(idx.shape[0])(table, idx)
    return out.reshape(*shape, D)


# speedup vs baseline: 2.3983x; 1.0995x over previous
import functools

import jax
import jax.numpy as jnp
from jax import lax
from jax.experimental import pallas as pl
from jax.experimental.pallas import tpu as pltpu
from jax.experimental.pallas import tpu_sc as plsc

NC = 2
NS = 16
NW = NC * NS
D = 1024
C = 32


@functools.lru_cache(maxsize=None)
def _make(B):
    bpw = B // NW
    nchunks = bpw // C
    mesh = plsc.VectorSubcoreMesh(core_axis_name="c", subcore_axis_name="s")

    @functools.partial(
        pl.kernel,
        mesh=mesh,
        out_type=jax.ShapeDtypeStruct((B, D), jnp.float32),
        scratch_types=[
            pltpu.VMEM((bpw,), jnp.int32),
            pltpu.VMEM((2, C, D), jnp.float32),
            pltpu.SemaphoreType.DMA,
            pltpu.SemaphoreType.DMA,
        ],
    )
    def gather_kernel(table_hbm, idx_hbm, out_hbm, idx_v, rows_v, gsem, ssem):
        wid = lax.axis_index("s") * NC + lax.axis_index("c")
        base = wid * bpw
        pltpu.sync_copy(idx_hbm.at[pl.ds(base, bpw)], idx_v)

        def gather(i, buf):
            return pltpu.make_async_copy(
                table_hbm.at[idx_v.at[pl.ds(i * C, C)]], rows_v.at[buf], gsem
            )

        def writeback(i, buf):
            return pltpu.make_async_copy(
                rows_v.at[buf], out_hbm.at[pl.ds(base + i * C, C)], ssem
            )

        gather(0, 0).start()

        def chunk(i, carry):
            buf = lax.rem(i, 2)
            nbuf = lax.rem(i + 1, 2)

            @pl.when(i + 1 < nchunks)
            def _prefetch():
                @pl.when(i >= 1)
                def _():
                    writeback(i - 1, nbuf).wait()

                gather(i + 1, nbuf).start()

            gather(i, buf).wait()
            writeback(i, buf).start()
            return carry

        lax.fori_loop(0, nchunks, chunk, 0, unroll=False)
        writeback(nchunks - 2, 0).wait()
        writeback(nchunks - 1, 1).wait()

    return gather_kernel


def kernel(input_batch, table):
    shape = input_batch.shape
    idx = input_batch.reshape(-1).astype(jnp.int32)
    out = _make(idx.shape[0])(table, idx)
    return out.reshape(*shape, D)


# speedup vs baseline: 2.4215x; 1.0097x over previous
import functools

import jax
import jax.numpy as jnp
from jax import lax
from jax.experimental import pallas as pl
from jax.experimental.pallas import tpu as pltpu
from jax.experimental.pallas import tpu_sc as plsc

NC = 2
NS = 16
NW = NC * NS
D = 1024
C = 32
NBUF = 3


@functools.lru_cache(maxsize=None)
def _make(B):
    bpw = B // NW
    nchunks = bpw // C
    mesh = plsc.VectorSubcoreMesh(core_axis_name="c", subcore_axis_name="s")

    @functools.partial(
        pl.kernel,
        mesh=mesh,
        out_type=jax.ShapeDtypeStruct((B, D), jnp.float32),
        scratch_types=[
            pltpu.VMEM((bpw,), jnp.int32),
            pltpu.VMEM((NBUF, C, D), jnp.float32),
            pltpu.SemaphoreType.DMA,
            pltpu.SemaphoreType.DMA,
        ],
    )
    def gather_kernel(table_hbm, idx_hbm, out_hbm, idx_v, rows_v, gsem, ssem):
        wid = lax.axis_index("s") * NC + lax.axis_index("c")
        base = wid * bpw
        pltpu.sync_copy(idx_hbm.at[pl.ds(base, bpw)], idx_v)

        def gather(i):
            return pltpu.make_async_copy(
                table_hbm.at[idx_v.at[pl.ds(i * C, C)]],
                rows_v.at[lax.rem(i, NBUF)],
                gsem,
            )

        def writeback(i):
            return pltpu.make_async_copy(
                rows_v.at[lax.rem(i, NBUF)],
                out_hbm.at[pl.ds(base + i * C, C)],
                ssem,
            )

        for j in range(NBUF - 1):
            gather(j).start()

        def chunk(i, carry):
            @pl.when(i + NBUF - 1 < nchunks)
            def _prefetch():
                @pl.when(i >= 1)
                def _():
                    writeback(i - 1).wait()

                gather(i + NBUF - 1).start()

            gather(i).wait()
            writeback(i).start()
            return carry

        lax.fori_loop(0, nchunks, chunk, 0, unroll=False)
        for j in range(NBUF):
            writeback(nchunks - NBUF + j).wait()

    return gather_kernel


def kernel(input_batch, table):
    shape = input_batch.shape
    idx = input_batch.reshape(-1).astype(jnp.int32)
    out = _make(idx.shape[0])(table, idx)
    return out.reshape(*shape, D)
